# baseline (speedup 1.0000x reference)
import functools

import jax
import jax.numpy as jnp
from jax import lax
from jax.experimental import pallas as pl
from jax.experimental.pallas import tpu as pltpu
from jax.experimental.pallas import tpu_sc as plsc


_NW = 32
_R = 16
_UNROLL = 8
_NXB = 4


def _sc_kernel_body(B, L, D, x_hbm, pos_hbm, out_hbm,
                    xb0, xb1, pb0, pb1, sl0, sl1, sp0, sp1, ss0, ss1):
    seq_per_w = L // _NW
    units = seq_per_w // _R
    wid = lax.axis_index("s") * 2 + lax.axis_index("c")
    seq0 = wid * seq_per_w

    xb = (xb0, xb1)
    pb = (pb0, pb1)
    sl = (sl0, sl1)
    sp = (sp0, sp1)
    ss = (ss0, ss1)

    def rows(u):
        return pl.ds(seq0 + u * _R, _R)

    def compute(u, j):
        k = j % 2
        pltpu.make_async_copy(x_hbm.at[:, rows(u)], xb[k], sl[k]).wait()
        pltpu.make_async_copy(pos_hbm.at[rows(u)], pb[k], sp[k]).wait()

        def add(r, carry):
            for b in range(B):
                for jj in range(D // 16):
                    s = pl.ds(jj * 16, 16)
                    xb[k][b, r, s] = xb[k][b, r, s] + pb[k][r, s]
            return carry

        lax.fori_loop(0, _R, add, 0)
        pltpu.async_copy(xb[k], out_hbm.at[:, rows(u)], ss[k])

    def super_iter(h, carry):
        for j in range(_UNROLL):
            u = h * _UNROLL + j
            k = j % 2

            @pl.when(u >= 2)
            def _():
                pltpu.make_async_copy(xb[k], out_hbm.at[:, rows(u - 2)],
                                      ss[k]).wait()

            pltpu.async_copy(x_hbm.at[:, rows(u)], xb[k], sl[k])
            pltpu.async_copy(pos_hbm.at[rows(u)], pb[k], sp[k])

            @pl.when(u >= 1)
            def _():
                compute(u - 1, j - 1)

        return carry

    lax.fori_loop(0, units // _UNROLL, super_iter, 0)
    compute(units - 1, _UNROLL - 1)
    for k in (0, 1):
        u = units - 2 + k
        pltpu.make_async_copy(xb[k], out_hbm.at[:, rows(u)], ss[k]).wait()


def _sc_add(x, pos_table):
    B, L, D = x.shape
    mesh = plsc.VectorSubcoreMesh(core_axis_name="c", subcore_axis_name="s")
    body = functools.partial(_sc_kernel_body, B, L, D)
    run = pl.kernel(
        body,
        mesh=mesh,
        out_type=jax.ShapeDtypeStruct((B, L, D), jnp.float32),
        scratch_types=[
            pltpu.VMEM((B, _R, D), jnp.float32),
            pltpu.VMEM((B, _R, D), jnp.float32),
            pltpu.VMEM((_R, D), jnp.float32),
            pltpu.VMEM((_R, D), jnp.float32),
            pltpu.SemaphoreType.DMA,
            pltpu.SemaphoreType.DMA,
            pltpu.SemaphoreType.DMA,
            pltpu.SemaphoreType.DMA,
            pltpu.SemaphoreType.DMA,
            pltpu.SemaphoreType.DMA,
        ],
    )
    return run(x, pos_table)


def _add_pos_kernel(x_ref, pos_ref, out_ref):
    out_ref[...] = x_ref[...] + pos_ref[...][None, :, :]


def _tc_add(x, pos_table):
    B, L, D = x.shape
    LB = 512
    return pl.pallas_call(
        _add_pos_kernel,
        grid=(L // LB,),
        in_specs=[
            pl.BlockSpec((B, LB, D), lambda i: (0, i, 0)),
            pl.BlockSpec((LB, D), lambda i: (i, 0)),
        ],
        out_specs=pl.BlockSpec((B, LB, D), lambda i: (0, i, 0)),
        out_shape=jax.ShapeDtypeStruct((B, L, D), x.dtype),
    )(x, pos_table)


def kernel(x, pos_table):
    return _sc_add(x, pos_table)


# speedup vs baseline: 1.5095x; 1.5095x over previous
import functools

import jax
import jax.numpy as jnp
from jax import lax
from jax.experimental import pallas as pl
from jax.experimental.pallas import tpu as pltpu
from jax.experimental.pallas import tpu_sc as plsc


_NW = 32
_R = 32
_UNROLL = 8


def _sc_kernel_body(B, L, D, x_hbm, pos_hbm, out_hbm,
                    xb0, xb1, pb0, pb1, sl0, sl1, sp0, sp1, ss0, ss1):
    seq_per_w = L // _NW
    nchunk = seq_per_w // _R
    units = nchunk * B
    wid = lax.axis_index("s") * 2 + lax.axis_index("c")
    seq0 = wid * seq_per_w

    xb = (xb0, xb1)
    pb = (pb0, pb1)
    sl = (sl0, sl1)
    sp = (sp0, sp1)
    ss = (ss0, ss1)

    def xrow(u, j):
        return (j % 4) * L + seq0 + (u // 4) * _R

    def prow(u):
        return seq0 + (u // 4) * _R

    def compute(u, j):
        j = j % _UNROLL
        k = j % 2
        p = j // 4
        pltpu.make_async_copy(x_hbm.at[pl.ds(xrow(u, j), _R)], xb[k],
                              sl[k]).wait()
        if j % 4 == 0:
            pltpu.make_async_copy(pos_hbm.at[pl.ds(prow(u), _R)], pb[p],
                                  sp[p]).wait()

        def add(r, carry):
            for jj in range(D // 16):
                s = pl.ds(jj * 16, 16)
                xb[k][r, s] = xb[k][r, s] + pb[p][r, s]
            return carry

        lax.fori_loop(0, _R, add, 0)
        pltpu.async_copy(xb[k], out_hbm.at[pl.ds(xrow(u, j), _R)], ss[k])

    def super_iter(h, carry):
        for j in range(_UNROLL):
            u = h * _UNROLL + j
            k = j % 2
            p = j // 4

            @pl.when(u >= 2)
            def _():
                pltpu.make_async_copy(
                    xb[k], out_hbm.at[pl.ds(xrow(u - 2, j - 2), _R)],
                    ss[k]).wait()

            pltpu.async_copy(x_hbm.at[pl.ds(xrow(u, j), _R)], xb[k],
                             sl[k])

            @pl.when(u >= 1)
            def _():
                compute(u - 1, j - 1)

            if j % 4 == 1:
                @pl.when(u + 4 < units)
                def _():
                    pltpu.async_copy(
                        pos_hbm.at[pl.ds(prow(u + 4), _R)], pb[1 - p],
                        sp[1 - p])

        return carry

    pltpu.async_copy(pos_hbm.at[pl.ds(seq0, _R)], pb[0], sp[0])
    lax.fori_loop(0, units // _UNROLL, super_iter, 0)
    compute(units - 1, _UNROLL - 1)
    for k in (0, 1):
        u = units - 2 + k
        pltpu.make_async_copy(xb[k], out_hbm.at[pl.ds(xrow(u, u), _R)],
                              ss[k]).wait()


def _sc_add(x, pos_table):
    B, L, D = x.shape
    mesh = plsc.VectorSubcoreMesh(core_axis_name="c", subcore_axis_name="s")
    body = functools.partial(_sc_kernel_body, B, L, D)
    run = pl.kernel(
        body,
        mesh=mesh,
        out_type=jax.ShapeDtypeStruct((B * L, D), jnp.float32),
        scratch_types=[
            pltpu.VMEM((_R, D), jnp.float32),
            pltpu.VMEM((_R, D), jnp.float32),
            pltpu.VMEM((_R, D), jnp.float32),
            pltpu.VMEM((_R, D), jnp.float32),
            pltpu.SemaphoreType.DMA,
            pltpu.SemaphoreType.DMA,
            pltpu.SemaphoreType.DMA,
            pltpu.SemaphoreType.DMA,
            pltpu.SemaphoreType.DMA,
            pltpu.SemaphoreType.DMA,
        ],
    )
    out = run(x.reshape(B * L, D), pos_table)
    return out.reshape(B, L, D)


def kernel(x, pos_table):
    return _sc_add(x, pos_table)
